# baseline (speedup 1.0000x reference)
import functools

import jax
import jax.numpy as jnp
from jax import lax
from jax.experimental import pallas as pl
from jax.experimental.pallas import tpu as pltpu
from jax.experimental.pallas import tpu_sc as plsc

_NC = 2
_NS = 16
_K = 128
_L = 16
_R = 2520
_NR = 4
_APAD = 2560
_BLK1 = 1008
_BLK2 = 504

_F32 = jnp.float32


def _dot(a, b):
    return jnp.dot(a, b, preferred_element_type=_F32)


def _mlp1_call(x, w1, b1, w2, b2):
    n, d = x.shape
    hn = w1.shape[1]
    do = w2.shape[1]

    def body(x_ref, w1r, b1r, w2r, b2r, y_ref):
        h = jnp.maximum(_dot(x_ref[...], w1r[...]) + b1r[...], 0.0)
        y_ref[...] = _dot(h, w2r[...]) + b2r[...]

    return pl.pallas_call(
        body,
        grid=(n // _BLK1,),
        in_specs=[
            pl.BlockSpec((_BLK1, d), lambda i: (i, 0)),
            pl.BlockSpec((d, hn), lambda i: (0, 0)),
            pl.BlockSpec((1, hn), lambda i: (0, 0)),
            pl.BlockSpec((hn, do), lambda i: (0, 0)),
            pl.BlockSpec((1, do), lambda i: (0, 0)),
        ],
        out_specs=pl.BlockSpec((_BLK1, do), lambda i: (i, 0)),
        out_shape=jax.ShapeDtypeStruct((n, do), _F32),
    )(x, w1, b1, w2, b2)


def _mlp2_call(x, accs, batch2, u, w2x, w2a, w2u, b1, w22, b2):
    n, d = x.shape
    nb = u.shape[0]
    hn = w2a.shape[0]
    bpr = _R // _BLK2

    def body(x_ref, acc_ref, b_ref, u_ref, w2xr, w2ar, w2ur, b1r,
             w22r, b2r, xo, gm, bc):
        i = pl.program_id(0)
        accsum = acc_ref[0]
        deg = accsum[:, hn:hn + 1]
        agg = accsum[:, :hn] / jnp.maximum(deg, 1.0)
        oh = (b_ref[...] == lax.broadcasted_iota(jnp.int32, (1, nb), 1))
        oh = oh.astype(_F32)
        t = _dot(u_ref[...], w2ur[...])
        h = (_dot(x_ref[...], w2xr[...]) + _dot(agg, w2ar[...])
             + _dot(oh, t) + b1r[...])
        xn = _dot(jnp.maximum(h, 0.0), w22r[...]) + b2r[...]
        xo[...] = xn

        @pl.when(i == 0)
        def _():
            gm[...] = jnp.zeros_like(gm)
            bc[...] = jnp.zeros_like(bc)

        gm[...] += lax.dot_general(oh, xn, (((0,), (0,)), ((), ())),
                                   preferred_element_type=_F32)
        bc[...] += jnp.broadcast_to(jnp.sum(oh, axis=0)[:, None],
                                    (nb, d))

    return pl.pallas_call(
        body,
        grid=(n // _BLK2,),
        in_specs=[
            pl.BlockSpec((_BLK2, d), lambda i: (i, 0)),
            pl.BlockSpec((1, _BLK2, d), lambda i: (i // bpr, i % bpr, 0)),
            pl.BlockSpec((_BLK2, 1), lambda i: (i, 0)),
            pl.BlockSpec((nb, d), lambda i: (0, 0)),
            pl.BlockSpec((d, hn), lambda i: (0, 0)),
            pl.BlockSpec((hn, hn), lambda i: (0, 0)),
            pl.BlockSpec((d, hn), lambda i: (0, 0)),
            pl.BlockSpec((1, hn), lambda i: (0, 0)),
            pl.BlockSpec((hn, d), lambda i: (0, 0)),
            pl.BlockSpec((1, d), lambda i: (0, 0)),
        ],
        out_specs=[
            pl.BlockSpec((_BLK2, d), lambda i: (i, 0)),
            pl.BlockSpec((nb, d), lambda i: (0, 0)),
            pl.BlockSpec((nb, d), lambda i: (0, 0)),
        ],
        out_shape=[
            jax.ShapeDtypeStruct((n, d), _F32),
            jax.ShapeDtypeStruct((nb, d), _F32),
            jax.ShapeDtypeStruct((nb, d), _F32),
        ],
    )(x, accs, batch2, u, w2x, w2a, w2u, b1, w22, b2)


def _global_call(u, gm_sum, bc, w1u, w1g, b1, w2, b2):
    nb, d = u.shape
    hg = w1u.shape[1]
    out = w2.shape[1]

    def body(u_ref, gm_ref, bc_ref, w1ur, w1gr, b1r, w2r, b2r, uo):
        gm = gm_ref[...] / jnp.maximum(bc_ref[...], 1.0)
        h = jnp.maximum(_dot(u_ref[...], w1ur[...]) + _dot(gm, w1gr[...])
                        + b1r[...], 0.0)
        uo[...] = _dot(h, w2r[...]) + b2r[...]

    return pl.pallas_call(
        body,
        in_specs=[
            pl.BlockSpec((nb, d), lambda: (0, 0)),
            pl.BlockSpec((nb, d), lambda: (0, 0)),
            pl.BlockSpec((nb, d), lambda: (0, 0)),
            pl.BlockSpec((d, hg), lambda: (0, 0)),
            pl.BlockSpec((d, hg), lambda: (0, 0)),
            pl.BlockSpec((1, hg), lambda: (0, 0)),
            pl.BlockSpec((hg, out), lambda: (0, 0)),
            pl.BlockSpec((1, out), lambda: (0, 0)),
        ],
        out_specs=pl.BlockSpec((nb, out), lambda: (0, 0)),
        out_shape=jax.ShapeDtypeStruct((nb, out), _F32),
    )(u, gm_sum, bc, w1u, w1g, b1, w2, b2)


@functools.lru_cache(maxsize=None)
def _edge_agg_kernel(d, ch):
    rpt = _APAD // _NS
    mesh = plsc.VectorSubcoreMesh(core_axis_name="c", subcore_axis_name="s",
                                  num_cores=_NC, num_subcores=_NS)

    def body(y_hbm, row_hbm, col_hbm, zeros_hbm, out_hbm,
             idx_row, idx_col, idx_loc, gbuf_a, gbuf_b, acc,
             sem_a, sem_b):
        cid = lax.axis_index("c")
        sid = lax.axis_index("s")
        r0 = sid * rpt
        pltpu.sync_copy(row_hbm.at[sid], idx_row)
        pltpu.sync_copy(col_hbm.at[sid], idx_col)

        for p in range(2):
            base = ((cid * 2 + p) * _R).astype(jnp.int32)
            pltpu.sync_copy(zeros_hbm, acc.at[pl.ds(r0, rpt)])

            @pl.loop(0, ch)
            def _remap(j):
                for l in range(_K // _L):
                    v = idx_col[j, pl.ds(l * _L, _L)] - base
                    ok = (v >= 0) & (v < _R)
                    idx_loc[j, pl.ds(l * _L, _L)] = jnp.where(ok, v, _R)

            plsc.subcore_barrier()

            pltpu.async_copy(y_hbm.at[idx_row.at[0]], gbuf_a, sem_a)
            pltpu.async_copy(y_hbm.at[idx_row.at[1]], gbuf_b, sem_b)

            @pl.loop(0, (ch - 2) // 2)
            def _chunk2(g):
                j0 = g * 2
                pltpu.make_async_copy(y_hbm.at[idx_row.at[j0]], gbuf_a,
                                      sem_a).wait()
                pltpu.sync_copy(gbuf_a, acc.at[idx_loc.at[j0]], add=True)
                pltpu.async_copy(y_hbm.at[idx_row.at[j0 + 2]], gbuf_a,
                                 sem_a)
                pltpu.make_async_copy(y_hbm.at[idx_row.at[j0 + 1]], gbuf_b,
                                      sem_b).wait()
                pltpu.sync_copy(gbuf_b, acc.at[idx_loc.at[j0 + 1]],
                                add=True)
                pltpu.async_copy(y_hbm.at[idx_row.at[j0 + 3]], gbuf_b,
                                 sem_b)

            pltpu.make_async_copy(y_hbm.at[idx_row.at[ch - 2]], gbuf_a,
                                  sem_a).wait()
            pltpu.make_async_copy(y_hbm.at[idx_row.at[ch - 1]], gbuf_b,
                                  sem_b).wait()

            plsc.subcore_barrier()
            pltpu.sync_copy(acc.at[pl.ds(r0, rpt)],
                            out_hbm.at[cid * 2 + p, pl.ds(r0, rpt)])
            plsc.subcore_barrier()

    return pl.kernel(
        body,
        out_type=jax.ShapeDtypeStruct((_NR, _APAD, d), _F32),
        mesh=mesh,
        scratch_types=[
            pltpu.VMEM((ch, _K), jnp.int32),
            pltpu.VMEM((ch, _K), jnp.int32),
            pltpu.VMEM((ch, _K), jnp.int32),
            pltpu.VMEM((_K, d), _F32),
            pltpu.VMEM((_K, d), _F32),
            pltpu.VMEM_SHARED((_APAD, d), _F32),
            pltpu.SemaphoreType.DMA,
            pltpu.SemaphoreType.DMA,
        ],
    )


def _edge_agg_call(y, row3, col3):
    d = y.shape[1]
    ch = row3.shape[1]
    rpt = _APAD // _NS
    f = _edge_agg_kernel(d, ch)
    return f(y, row3, col3, jnp.zeros((rpt, d), _F32))


def kernel(x, edge_index, u, batch, params):
    n, d = x.shape
    e = edge_index.shape[1]
    row = edge_index[0].astype(jnp.int32)
    col = edge_index[1].astype(jnp.int32)

    npad = _NR * _R
    grp = _NS * _K
    ep = -(-e // (2 * grp)) * (2 * grp) + 2 * grp

    row3 = jnp.concatenate([row, jnp.zeros((ep - e,), jnp.int32)])
    row3 = row3.reshape(_NS, ep // grp, _K)
    col3 = jnp.concatenate([col, jnp.full((ep - e,), n, jnp.int32)])
    col3 = col3.reshape(_NS, ep // grp, _K)

    xp = jnp.concatenate([x, jnp.zeros((npad - n, d), _F32)])
    batch2 = jnp.concatenate([batch.astype(jnp.int32),
                              jnp.full((npad - n,), 16, jnp.int32)])
    batch2 = batch2.reshape(npad, 1)

    hn = params[0]['n1_W2'].shape[1]

    def stack(f):
        return jnp.stack([f(p) for p in params])

    xs = (
        stack(lambda p: p['n1_W1']),
        stack(lambda p: p['n1_b1'].reshape(1, -1)),
        stack(lambda p: jnp.zeros((hn, d), _F32).at[:, :hn].set(p['n1_W2'])),
        stack(lambda p: jnp.zeros((1, d), _F32)
              .at[0, :hn].set(p['n1_b2']).at[0, hn].set(1.0)),
        stack(lambda p: p['n2_W1'][:d]),
        stack(lambda p: p['n2_W1'][d:d + hn]),
        stack(lambda p: p['n2_W1'][d + hn:]),
        stack(lambda p: p['n2_b1'].reshape(1, -1)),
        stack(lambda p: p['n2_W2']),
        stack(lambda p: p['n2_b2'].reshape(1, -1)),
        stack(lambda p: p['g_W1'][:d]),
        stack(lambda p: p['g_W1'][d:]),
        stack(lambda p: p['g_b1'].reshape(1, -1)),
        stack(lambda p: p['g_W2']),
        stack(lambda p: p['g_b2'].reshape(1, -1)),
    )

    def layer(carry, pw):
        x, u = carry
        (w1, b1, w2p, b2p, w2x, w2a, w2u, nb1, w22, nb2,
         gw1u, gw1g, gb1, gw2, gb2) = pw
        y = _mlp1_call(x, w1, b1, w2p, b2p)
        accs = _edge_agg_call(y, row3, col3)
        x, gm_sum, bc = _mlp2_call(x, accs, batch2, u,
                                   w2x, w2a, w2u, nb1, w22, nb2)
        u = _global_call(u, gm_sum, bc, gw1u, gw1g, gb1, gw2, gb2)
        return (x, u), None

    (xp, u), _ = lax.scan(layer, (xp, u), xs)
    return (xp[:n], u)


# speedup vs baseline: 1.8574x; 1.8574x over previous
import functools

import jax
import jax.numpy as jnp
from jax import lax
from jax.experimental import pallas as pl
from jax.experimental.pallas import tpu as pltpu
from jax.experimental.pallas import tpu_sc as plsc

_NC = 2
_NS = 16
_K = 128
_L = 16
_R = 2520
_NR = 4
_APAD = 2560
_BLK1 = 1008
_BLK2 = 504

_F32 = jnp.float32


def _dot(a, b):
    return jnp.dot(a, b, preferred_element_type=_F32)


def _mlp1_call(x, w1, b1, w2, b2):
    n, d = x.shape
    hn = w1.shape[1]
    do = w2.shape[1]

    def body(x_ref, w1r, b1r, w2r, b2r, y_ref):
        h = jnp.maximum(_dot(x_ref[...], w1r[...]) + b1r[...], 0.0)
        y_ref[...] = _dot(h, w2r[...]) + b2r[...]

    return pl.pallas_call(
        body,
        grid=(n // _BLK1,),
        in_specs=[
            pl.BlockSpec((_BLK1, d), lambda i: (i, 0)),
            pl.BlockSpec((d, hn), lambda i: (0, 0)),
            pl.BlockSpec((1, hn), lambda i: (0, 0)),
            pl.BlockSpec((hn, do), lambda i: (0, 0)),
            pl.BlockSpec((1, do), lambda i: (0, 0)),
        ],
        out_specs=pl.BlockSpec((_BLK1, do), lambda i: (i, 0)),
        out_shape=jax.ShapeDtypeStruct((n, do), _F32),
    )(x, w1, b1, w2, b2)


def _mlp2_call(x, accs, batch2, u, w2x, w2a, w2u, b1, w22, b2):
    n, d = x.shape
    nb = u.shape[0]
    hn = w2a.shape[0]
    bpr = _R // _BLK2

    def body(x_ref, acc_ref, b_ref, u_ref, w2xr, w2ar, w2ur, b1r,
             w22r, b2r, xo, gm, bc):
        i = pl.program_id(0)
        accsum = acc_ref[0]
        deg = accsum[:, hn:hn + 1]
        agg = accsum[:, :hn] / jnp.maximum(deg, 1.0)
        oh = (b_ref[...] == lax.broadcasted_iota(jnp.int32, (1, nb), 1))
        oh = oh.astype(_F32)
        t = _dot(u_ref[...], w2ur[...])
        h = (_dot(x_ref[...], w2xr[...]) + _dot(agg, w2ar[...])
             + _dot(oh, t) + b1r[...])
        xn = _dot(jnp.maximum(h, 0.0), w22r[...]) + b2r[...]
        xo[...] = xn

        @pl.when(i == 0)
        def _():
            gm[...] = jnp.zeros_like(gm)
            bc[...] = jnp.zeros_like(bc)

        gm[...] += lax.dot_general(oh, xn, (((0,), (0,)), ((), ())),
                                   preferred_element_type=_F32)
        bc[...] += jnp.broadcast_to(jnp.sum(oh, axis=0)[:, None],
                                    (nb, d))

    return pl.pallas_call(
        body,
        grid=(n // _BLK2,),
        in_specs=[
            pl.BlockSpec((_BLK2, d), lambda i: (i, 0)),
            pl.BlockSpec((1, _BLK2, d), lambda i: (i // bpr, i % bpr, 0)),
            pl.BlockSpec((_BLK2, 1), lambda i: (i, 0)),
            pl.BlockSpec((nb, d), lambda i: (0, 0)),
            pl.BlockSpec((d, hn), lambda i: (0, 0)),
            pl.BlockSpec((hn, hn), lambda i: (0, 0)),
            pl.BlockSpec((d, hn), lambda i: (0, 0)),
            pl.BlockSpec((1, hn), lambda i: (0, 0)),
            pl.BlockSpec((hn, d), lambda i: (0, 0)),
            pl.BlockSpec((1, d), lambda i: (0, 0)),
        ],
        out_specs=[
            pl.BlockSpec((_BLK2, d), lambda i: (i, 0)),
            pl.BlockSpec((nb, d), lambda i: (0, 0)),
            pl.BlockSpec((nb, d), lambda i: (0, 0)),
        ],
        out_shape=[
            jax.ShapeDtypeStruct((n, d), _F32),
            jax.ShapeDtypeStruct((nb, d), _F32),
            jax.ShapeDtypeStruct((nb, d), _F32),
        ],
    )(x, accs, batch2, u, w2x, w2a, w2u, b1, w22, b2)


def _global_call(u, gm_sum, bc, w1u, w1g, b1, w2, b2):
    nb, d = u.shape
    hg = w1u.shape[1]
    out = w2.shape[1]

    def body(u_ref, gm_ref, bc_ref, w1ur, w1gr, b1r, w2r, b2r, uo):
        gm = gm_ref[...] / jnp.maximum(bc_ref[...], 1.0)
        h = jnp.maximum(_dot(u_ref[...], w1ur[...]) + _dot(gm, w1gr[...])
                        + b1r[...], 0.0)
        uo[...] = _dot(h, w2r[...]) + b2r[...]

    return pl.pallas_call(
        body,
        in_specs=[
            pl.BlockSpec((nb, d), lambda: (0, 0)),
            pl.BlockSpec((nb, d), lambda: (0, 0)),
            pl.BlockSpec((nb, d), lambda: (0, 0)),
            pl.BlockSpec((d, hg), lambda: (0, 0)),
            pl.BlockSpec((d, hg), lambda: (0, 0)),
            pl.BlockSpec((1, hg), lambda: (0, 0)),
            pl.BlockSpec((hg, out), lambda: (0, 0)),
            pl.BlockSpec((1, out), lambda: (0, 0)),
        ],
        out_specs=pl.BlockSpec((nb, out), lambda: (0, 0)),
        out_shape=jax.ShapeDtypeStruct((nb, out), _F32),
    )(u, gm_sum, bc, w1u, w1g, b1, w2, b2)


@functools.lru_cache(maxsize=None)
def _edge_agg_kernel(d, ch):
    rpt = _APAD // _NS
    mesh = plsc.VectorSubcoreMesh(core_axis_name="c", subcore_axis_name="s",
                                  num_cores=_NC, num_subcores=_NS)

    def body(y_hbm, row_hbm, col_hbm, zeros_hbm, out_hbm,
             idx_row, idx_col, idx_loc, gbuf_a, acc, sem_a):
        cid = lax.axis_index("c")
        sid = lax.axis_index("s")
        r0 = sid * rpt
        pltpu.sync_copy(row_hbm.at[sid], idx_row)
        pltpu.sync_copy(col_hbm.at[sid], idx_col)

        for p in range(2):
            base = ((cid * 2 + p) * _R).astype(jnp.int32)
            pltpu.sync_copy(zeros_hbm, acc.at[pl.ds(r0, rpt)])

            @pl.loop(0, ch)
            def _remap(j):
                for l in range(_K // _L):
                    v = idx_col[j, pl.ds(l * _L, _L)] - base
                    ok = (v >= 0) & (v < _R)
                    idx_loc[j, pl.ds(l * _L, _L)] = jnp.where(ok, v, _R)

            plsc.subcore_barrier()

            @pl.loop(0, ch)
            def _chunk(j):
                pltpu.async_copy(y_hbm.at[idx_row.at[j]], gbuf_a,
                                 sem_a).wait()
                pltpu.sync_copy(gbuf_a, acc.at[idx_loc.at[j]], add=True)

            plsc.subcore_barrier()
            pltpu.sync_copy(acc.at[pl.ds(r0, rpt)],
                            out_hbm.at[cid * 2 + p, pl.ds(r0, rpt)])
            plsc.subcore_barrier()

    return pl.kernel(
        body,
        out_type=jax.ShapeDtypeStruct((_NR, _APAD, d), _F32),
        mesh=mesh,
        scratch_types=[
            pltpu.VMEM((ch, _K), jnp.int32),
            pltpu.VMEM((ch, _K), jnp.int32),
            pltpu.VMEM((ch, _K), jnp.int32),
            pltpu.VMEM((_K, d), _F32),
            pltpu.VMEM_SHARED((_APAD, d), _F32),
            pltpu.SemaphoreType.DMA,
        ],
    )


def _edge_agg_call(y, row3, col3):
    d = y.shape[1]
    ch = row3.shape[1]
    rpt = _APAD // _NS
    f = _edge_agg_kernel(d, ch)
    return f(y, row3, col3, jnp.zeros((rpt, d), _F32))


def kernel(x, edge_index, u, batch, params):
    n, d = x.shape
    e = edge_index.shape[1]
    row = edge_index[0].astype(jnp.int32)
    col = edge_index[1].astype(jnp.int32)

    npad = _NR * _R
    grp = _NS * _K
    ep = -(-e // grp) * grp

    row3 = jnp.concatenate([row, jnp.zeros((ep - e,), jnp.int32)])
    row3 = row3.reshape(_NS, ep // grp, _K)
    col3 = jnp.concatenate([col, jnp.full((ep - e,), n, jnp.int32)])
    col3 = col3.reshape(_NS, ep // grp, _K)

    xp = jnp.concatenate([x, jnp.zeros((npad - n, d), _F32)])
    batch2 = jnp.concatenate([batch.astype(jnp.int32),
                              jnp.full((npad - n,), 16, jnp.int32)])
    batch2 = batch2.reshape(npad, 1)

    hn = params[0]['n1_W2'].shape[1]

    def stack(f):
        return jnp.stack([f(p) for p in params])

    xs = (
        stack(lambda p: p['n1_W1']),
        stack(lambda p: p['n1_b1'].reshape(1, -1)),
        stack(lambda p: jnp.zeros((hn, d), _F32).at[:, :hn].set(p['n1_W2'])),
        stack(lambda p: jnp.zeros((1, d), _F32)
              .at[0, :hn].set(p['n1_b2']).at[0, hn].set(1.0)),
        stack(lambda p: p['n2_W1'][:d]),
        stack(lambda p: p['n2_W1'][d:d + hn]),
        stack(lambda p: p['n2_W1'][d + hn:]),
        stack(lambda p: p['n2_b1'].reshape(1, -1)),
        stack(lambda p: p['n2_W2']),
        stack(lambda p: p['n2_b2'].reshape(1, -1)),
        stack(lambda p: p['g_W1'][:d]),
        stack(lambda p: p['g_W1'][d:]),
        stack(lambda p: p['g_b1'].reshape(1, -1)),
        stack(lambda p: p['g_W2']),
        stack(lambda p: p['g_b2'].reshape(1, -1)),
    )

    def layer(carry, pw):
        x, u = carry
        (w1, b1, w2p, b2p, w2x, w2a, w2u, nb1, w22, nb2,
         gw1u, gw1g, gb1, gw2, gb2) = pw
        y = _mlp1_call(x, w1, b1, w2p, b2p)
        accs = _edge_agg_call(y, row3, col3)
        x, gm_sum, bc = _mlp2_call(x, accs, batch2, u,
                                   w2x, w2a, w2u, nb1, w22, nb2)
        u = _global_call(u, gm_sum, bc, gw1u, gw1g, gb1, gw2, gb2)
        return (x, u), None

    (xp, u), _ = lax.scan(layer, (xp, u), xs)
    return (xp[:n], u)


# speedup vs baseline: 3.4477x; 1.8562x over previous
import functools

import jax
import jax.numpy as jnp
from jax import lax
from jax.experimental import pallas as pl
from jax.experimental.pallas import tpu as pltpu
from jax.experimental.pallas import tpu_sc as plsc

_NC = 2
_NS = 16
_K = 128
_L = 16
_R = 5040
_APAD = 5120
_BLK1 = 1008
_BLK2 = 504

_F32 = jnp.float32

_UNPACK_COL = tuple(range(128))


def _dot(a, b):
    return jnp.dot(a, b, preferred_element_type=_F32)


def _mlp1_call(x, w1, b1, w2, b2):
    n, d = x.shape
    hn = w1.shape[1]
    do = w2.shape[1]

    def body(x_ref, w1r, b1r, w2r, b2r, y_ref):
        h = jnp.maximum(_dot(x_ref[...], w1r[...]) + b1r[...], 0.0)
        y_ref[...] = _dot(h, w2r[...]) + b2r[...]

    return pl.pallas_call(
        body,
        grid=(n // _BLK1,),
        in_specs=[
            pl.BlockSpec((_BLK1, d), lambda i: (i, 0)),
            pl.BlockSpec((d, hn), lambda i: (0, 0)),
            pl.BlockSpec((1, hn), lambda i: (0, 0)),
            pl.BlockSpec((hn, do), lambda i: (0, 0)),
            pl.BlockSpec((1, do), lambda i: (0, 0)),
        ],
        out_specs=pl.BlockSpec((_BLK1, do), lambda i: (i, 0)),
        out_shape=jax.ShapeDtypeStruct((n, do), _F32),
    )(x, w1, b1, w2, b2)


def _mlp2_call(x, accs, batch2, u, w2x, w2ax, w2u, b1, w22, b2):
    n, d = x.shape
    nb = u.shape[0]
    hn = w22.shape[0]
    bpr = _R // _BLK2
    dpos = 64

    def body(x_ref, acc_ref, b_ref, u_ref, w2xr, w2axr, w2ur, b1r,
             w22r, b2r, xo, gm, bc):
        i = pl.program_id(0)
        accp = acc_ref[0].astype(_F32)
        deg = accp[:, dpos:dpos + 1]
        agg = accp / jnp.maximum(deg, 1.0)
        oh = (b_ref[...] == lax.broadcasted_iota(jnp.int32, (1, nb), 1))
        oh = oh.astype(_F32)
        t = _dot(u_ref[...], w2ur[...])
        h = (_dot(x_ref[...], w2xr[...]) + _dot(agg, w2axr[...])
             + _dot(oh, t) + b1r[...])
        xn = _dot(jnp.maximum(h, 0.0), w22r[...]) + b2r[...]
        xo[...] = xn

        @pl.when(i == 0)
        def _():
            gm[...] = jnp.zeros_like(gm)
            bc[...] = jnp.zeros_like(bc)

        gm[...] += lax.dot_general(oh, xn, (((0,), (0,)), ((), ())),
                                   preferred_element_type=_F32)
        bc[...] += jnp.broadcast_to(jnp.sum(oh, axis=0)[:, None],
                                    (nb, d))

    return pl.pallas_call(
        body,
        grid=(n // _BLK2,),
        in_specs=[
            pl.BlockSpec((_BLK2, d), lambda i: (i, 0)),
            pl.BlockSpec((1, _BLK2, d), lambda i: (i // bpr, i % bpr, 0)),
            pl.BlockSpec((_BLK2, 1), lambda i: (i, 0)),
            pl.BlockSpec((nb, d), lambda i: (0, 0)),
            pl.BlockSpec((d, hn), lambda i: (0, 0)),
            pl.BlockSpec((d, hn), lambda i: (0, 0)),
            pl.BlockSpec((d, hn), lambda i: (0, 0)),
            pl.BlockSpec((1, hn), lambda i: (0, 0)),
            pl.BlockSpec((hn, d), lambda i: (0, 0)),
            pl.BlockSpec((1, d), lambda i: (0, 0)),
        ],
        out_specs=[
            pl.BlockSpec((_BLK2, d), lambda i: (i, 0)),
            pl.BlockSpec((nb, d), lambda i: (0, 0)),
            pl.BlockSpec((nb, d), lambda i: (0, 0)),
        ],
        out_shape=[
            jax.ShapeDtypeStruct((n, d), _F32),
            jax.ShapeDtypeStruct((nb, d), _F32),
            jax.ShapeDtypeStruct((nb, d), _F32),
        ],
    )(x, accs, batch2, u, w2x, w2ax, w2u, b1, w22, b2)


def _global_call(u, gm_sum, bc, w1u, w1g, b1, w2, b2):
    nb, d = u.shape
    hg = w1u.shape[1]
    out = w2.shape[1]

    def body(u_ref, gm_ref, bc_ref, w1ur, w1gr, b1r, w2r, b2r, uo):
        gm = gm_ref[...] / jnp.maximum(bc_ref[...], 1.0)
        h = jnp.maximum(_dot(u_ref[...], w1ur[...]) + _dot(gm, w1gr[...])
                        + b1r[...], 0.0)
        uo[...] = _dot(h, w2r[...]) + b2r[...]

    return pl.pallas_call(
        body,
        in_specs=[
            pl.BlockSpec((nb, d), lambda: (0, 0)),
            pl.BlockSpec((nb, d), lambda: (0, 0)),
            pl.BlockSpec((nb, d), lambda: (0, 0)),
            pl.BlockSpec((d, hg), lambda: (0, 0)),
            pl.BlockSpec((d, hg), lambda: (0, 0)),
            pl.BlockSpec((1, hg), lambda: (0, 0)),
            pl.BlockSpec((hg, out), lambda: (0, 0)),
            pl.BlockSpec((1, out), lambda: (0, 0)),
        ],
        out_specs=pl.BlockSpec((nb, out), lambda: (0, 0)),
        out_shape=jax.ShapeDtypeStruct((nb, out), _F32),
    )(u, gm_sum, bc, w1u, w1g, b1, w2, b2)


@functools.lru_cache(maxsize=None)
def _edge_agg_kernel(d, ch):
    rpt = _APAD // _NS
    cvr = 64
    mesh = plsc.VectorSubcoreMesh(core_axis_name="c", subcore_axis_name="s",
                                  num_cores=_NC, num_subcores=_NS)

    def body(y_hbm, row_hbm, col_hbm, zeros_hbm, out_hbm,
             idx_row, idx_col, idx_loc, gbuf, fbuf, obuf, acc, sem):
        cid = lax.axis_index("c")
        sid = lax.axis_index("s")
        r0 = sid * rpt
        base = (cid * _R).astype(jnp.int32)
        pltpu.sync_copy(zeros_hbm, acc.at[pl.ds(r0, rpt)])
        pltpu.sync_copy(row_hbm.at[sid], idx_row)
        pltpu.sync_copy(col_hbm.at[sid], idx_col)

        @pl.loop(0, ch)
        def _remap(j):
            for l in range(_K // _L):
                v = idx_col[j, pl.ds(l * _L, _L)] - base
                ok = (v >= 0) & (v < _R)
                idx_loc[j, pl.ds(l * _L, _L)] = jnp.where(ok, v, _R)

        plsc.subcore_barrier()

        @pl.loop(0, ch)
        def _chunk(j):
            pltpu.async_copy(y_hbm.at[idx_row.at[j]], gbuf, sem).wait()
            pltpu.sync_copy(gbuf, acc.at[idx_loc.at[j]], add=True)

        plsc.subcore_barrier()

        for q in range(rpt // cvr):
            pltpu.sync_copy(acc.at[pl.ds(r0 + q * cvr, cvr)], fbuf)

            @pl.loop(0, cvr)
            def _cv(r):
                for g in range(d // 32):
                    a = fbuf[r, pl.ds(g * 32, _L)]
                    b = fbuf[r, pl.ds(g * 32 + _L, _L)]
                    pk = plsc.pack(a, b, format=plsc.PackFormat.INTERLEAVED)
                    obuf[r, pl.ds(g * 32, 32)] = pk

            pltpu.sync_copy(obuf,
                            out_hbm.at[cid, pl.ds(r0 + q * cvr, cvr)])

    return pl.kernel(
        body,
        out_type=jax.ShapeDtypeStruct((_NC, _APAD, d), jnp.bfloat16),
        compiler_params=pltpu.CompilerParams(needs_layout_passes=False),
        mesh=mesh,
        scratch_types=[
            pltpu.VMEM((ch, _K), jnp.int32),
            pltpu.VMEM((ch, _K), jnp.int32),
            pltpu.VMEM((ch, _K), jnp.int32),
            pltpu.VMEM((_K, d), _F32),
            pltpu.VMEM((cvr, d), _F32),
            pltpu.VMEM((cvr, d), jnp.bfloat16),
            pltpu.VMEM_SHARED((_APAD, d), _F32),
            pltpu.SemaphoreType.DMA,
        ],
    )


def _edge_agg_call(y, row3, col3):
    d = y.shape[1]
    ch = row3.shape[1]
    rpt = _APAD // _NS
    f = _edge_agg_kernel(d, ch)
    return f(y, row3, col3, jnp.zeros((rpt, d), _F32))


def kernel(x, edge_index, u, batch, params):
    n, d = x.shape
    e = edge_index.shape[1]
    row = edge_index[0].astype(jnp.int32)
    col = edge_index[1].astype(jnp.int32)

    npad = _NC * _R
    grp = _NS * _K
    ep = -(-e // grp) * grp

    row3 = jnp.concatenate([row, jnp.zeros((ep - e,), jnp.int32)])
    row3 = row3.reshape(_NS, ep // grp, _K)
    col3 = jnp.concatenate([col, jnp.full((ep - e,), n, jnp.int32)])
    col3 = col3.reshape(_NS, ep // grp, _K)

    xp = jnp.concatenate([x, jnp.zeros((npad - n, d), _F32)])
    batch2 = jnp.concatenate([batch.astype(jnp.int32),
                              jnp.full((npad - n,), 16, jnp.int32)])
    batch2 = batch2.reshape(npad, 1)

    hn = params[0]['n1_W2'].shape[1]

    def stack(f):
        return jnp.stack([f(p) for p in params])

    xs = (
        stack(lambda p: p['n1_W1']),
        stack(lambda p: p['n1_b1'].reshape(1, -1)),
        stack(lambda p: jnp.zeros((hn, d), _F32).at[:, :hn].set(p['n1_W2'])),
        stack(lambda p: jnp.zeros((1, d), _F32)
              .at[0, :hn].set(p['n1_b2']).at[0, hn].set(1.0)),
        stack(lambda p: p['n2_W1'][:d]),
        stack(lambda p: jnp.concatenate(
            [p['n2_W1'][d:d + hn],
             jnp.zeros((d - hn, hn), _F32)])[jnp.asarray(_UNPACK_COL)]),
        stack(lambda p: p['n2_W1'][d + hn:]),
        stack(lambda p: p['n2_b1'].reshape(1, -1)),
        stack(lambda p: p['n2_W2']),
        stack(lambda p: p['n2_b2'].reshape(1, -1)),
        stack(lambda p: p['g_W1'][:d]),
        stack(lambda p: p['g_W1'][d:]),
        stack(lambda p: p['g_b1'].reshape(1, -1)),
        stack(lambda p: p['g_W2']),
        stack(lambda p: p['g_b2'].reshape(1, -1)),
    )

    def layer(carry, pw):
        x, u = carry
        (w1, b1, w2p, b2p, w2x, w2a, w2u, nb1, w22, nb2,
         gw1u, gw1g, gb1, gw2, gb2) = pw
        y = _mlp1_call(x, w1, b1, w2p, b2p)
        accs = _edge_agg_call(y, row3, col3)
        x, gm_sum, bc = _mlp2_call(x, accs, batch2, u,
                                   w2x, w2a, w2u, nb1, w22, nb2)
        u = _global_call(u, gm_sum, bc, gw1u, gw1g, gb1, gw2, gb2)
        return (x, u), None

    (xp, u), _ = lax.scan(layer, (xp, u), xs)
    return (xp[:n], u)


# speedup vs baseline: 3.4725x; 1.0072x over previous
import functools

import jax
import jax.numpy as jnp
from jax import lax
from jax.experimental import pallas as pl
from jax.experimental.pallas import tpu as pltpu
from jax.experimental.pallas import tpu_sc as plsc

_NC = 2
_NS = 16
_K = 128
_L = 16
_R = 5040
_APAD = 5120
_BLK1 = 1008
_BLK2 = 504

_F32 = jnp.float32

_PACK_LO = tuple(32 * (w // 16) + w % 16 for w in range(64))
_PACK_HI = tuple(c + 16 for c in _PACK_LO)
_DEG_W = _PACK_LO.index(64)


def _dot(a, b):
    return jnp.dot(a, b, preferred_element_type=_F32)


def _mlp1_call(x, w1, b1, w2, b2):
    n, d = x.shape
    hn = w1.shape[1]
    do = w2.shape[1]

    def body(x_ref, w1r, b1r, w2r, b2r, y_ref):
        h = jnp.maximum(_dot(x_ref[...], w1r[...]) + b1r[...], 0.0)
        y_ref[...] = _dot(h, w2r[...]) + b2r[...]

    return pl.pallas_call(
        body,
        grid=(n // _BLK1,),
        in_specs=[
            pl.BlockSpec((_BLK1, d), lambda i: (i, 0)),
            pl.BlockSpec((d, hn), lambda i: (0, 0)),
            pl.BlockSpec((1, hn), lambda i: (0, 0)),
            pl.BlockSpec((hn, do), lambda i: (0, 0)),
            pl.BlockSpec((1, do), lambda i: (0, 0)),
        ],
        out_specs=pl.BlockSpec((_BLK1, do), lambda i: (i, 0)),
        out_shape=jax.ShapeDtypeStruct((n, do), _F32),
    )(x, w1, b1, w2, b2)


def _mlp2_call(x, accs, batch2, u, w2x, w2axl, w2axh, w2u, b1, w22, b2):
    n, d = x.shape
    nb = u.shape[0]
    hn = w22.shape[0]
    bpr = _R // _BLK2

    def body(x_ref, acc_ref, b_ref, u_ref, w2xr, w2axlr, w2axhr, w2ur,
             b1r, w22r, b2r, xo, gm, bc):
        i = pl.program_id(0)
        wrd = acc_ref[0]
        lo_f = lax.bitcast_convert_type(wrd << 16, _F32)
        hi_f = lax.bitcast_convert_type(wrd & jnp.int32(-65536), _F32)
        rec = 1.0 / jnp.maximum(lo_f[:, _DEG_W:_DEG_W + 1], 1.0)
        oh = (b_ref[...] == lax.broadcasted_iota(jnp.int32, (1, nb), 1))
        oh = oh.astype(_F32)
        t = _dot(u_ref[...], w2ur[...])
        h = (_dot(x_ref[...], w2xr[...]) + _dot(lo_f * rec, w2axlr[...])
             + _dot(hi_f * rec, w2axhr[...])
             + _dot(oh, t) + b1r[...])
        xn = _dot(jnp.maximum(h, 0.0), w22r[...]) + b2r[...]
        xo[...] = xn

        @pl.when(i == 0)
        def _():
            gm[...] = jnp.zeros_like(gm)
            bc[...] = jnp.zeros_like(bc)

        gm[...] += lax.dot_general(oh, xn, (((0,), (0,)), ((), ())),
                                   preferred_element_type=_F32)
        bc[...] += jnp.broadcast_to(jnp.sum(oh, axis=0)[:, None],
                                    (nb, d))

    return pl.pallas_call(
        body,
        grid=(n // _BLK2,),
        in_specs=[
            pl.BlockSpec((_BLK2, d), lambda i: (i, 0)),
            pl.BlockSpec((1, _BLK2, d // 2),
                         lambda i: (i // bpr, i % bpr, 0)),
            pl.BlockSpec((_BLK2, 1), lambda i: (i, 0)),
            pl.BlockSpec((nb, d), lambda i: (0, 0)),
            pl.BlockSpec((d, hn), lambda i: (0, 0)),
            pl.BlockSpec((d // 2, hn), lambda i: (0, 0)),
            pl.BlockSpec((d // 2, hn), lambda i: (0, 0)),
            pl.BlockSpec((d, hn), lambda i: (0, 0)),
            pl.BlockSpec((1, hn), lambda i: (0, 0)),
            pl.BlockSpec((hn, d), lambda i: (0, 0)),
            pl.BlockSpec((1, d), lambda i: (0, 0)),
        ],
        out_specs=[
            pl.BlockSpec((_BLK2, d), lambda i: (i, 0)),
            pl.BlockSpec((nb, d), lambda i: (0, 0)),
            pl.BlockSpec((nb, d), lambda i: (0, 0)),
        ],
        out_shape=[
            jax.ShapeDtypeStruct((n, d), _F32),
            jax.ShapeDtypeStruct((nb, d), _F32),
            jax.ShapeDtypeStruct((nb, d), _F32),
        ],
    )(x, accs, batch2, u, w2x, w2axl, w2axh, w2u, b1, w22, b2)


def _global_call(u, gm_sum, bc, w1u, w1g, b1, w2, b2):
    nb, d = u.shape
    hg = w1u.shape[1]
    out = w2.shape[1]

    def body(u_ref, gm_ref, bc_ref, w1ur, w1gr, b1r, w2r, b2r, uo):
        gm = gm_ref[...] / jnp.maximum(bc_ref[...], 1.0)
        h = jnp.maximum(_dot(u_ref[...], w1ur[...]) + _dot(gm, w1gr[...])
                        + b1r[...], 0.0)
        uo[...] = _dot(h, w2r[...]) + b2r[...]

    return pl.pallas_call(
        body,
        in_specs=[
            pl.BlockSpec((nb, d), lambda: (0, 0)),
            pl.BlockSpec((nb, d), lambda: (0, 0)),
            pl.BlockSpec((nb, d), lambda: (0, 0)),
            pl.BlockSpec((d, hg), lambda: (0, 0)),
            pl.BlockSpec((d, hg), lambda: (0, 0)),
            pl.BlockSpec((1, hg), lambda: (0, 0)),
            pl.BlockSpec((hg, out), lambda: (0, 0)),
            pl.BlockSpec((1, out), lambda: (0, 0)),
        ],
        out_specs=pl.BlockSpec((nb, out), lambda: (0, 0)),
        out_shape=jax.ShapeDtypeStruct((nb, out), _F32),
    )(u, gm_sum, bc, w1u, w1g, b1, w2, b2)


@functools.lru_cache(maxsize=None)
def _edge_agg_kernel(d, ch):
    rpt = _APAD // _NS
    cvr = 64
    mesh = plsc.VectorSubcoreMesh(core_axis_name="c", subcore_axis_name="s",
                                  num_cores=_NC, num_subcores=_NS)

    def body(y_hbm, row_hbm, col_hbm, zeros_hbm, out_hbm,
             idx_row, idx_col, gbuf, fbuf, obuf, acc, sem):
        cid = lax.axis_index("c")
        sid = lax.axis_index("s")
        r0 = sid * rpt
        base = (cid * _R).astype(jnp.int32)
        pltpu.sync_copy(zeros_hbm, acc.at[pl.ds(r0, rpt)])
        pltpu.sync_copy(row_hbm.at[sid], idx_row)
        pltpu.sync_copy(col_hbm.at[sid], idx_col)

        @pl.loop(0, ch)
        def _remap(j):
            for l in range(_K // _L):
                v = idx_col[j, pl.ds(l * _L, _L)] - base
                ok = (v >= 0) & (v < _R)
                idx_col[j, pl.ds(l * _L, _L)] = jnp.where(ok, v, _R)

        plsc.subcore_barrier()

        @pl.loop(0, ch)
        def _chunk(j):
            pltpu.async_copy(y_hbm.at[idx_row.at[j]], gbuf, sem).wait()
            pltpu.sync_copy(gbuf, acc.at[idx_col.at[j]], add=True)

        plsc.subcore_barrier()

        for q in range(rpt // cvr):
            pltpu.sync_copy(acc.at[pl.ds(r0 + q * cvr, cvr)], fbuf)

            @pl.loop(0, cvr)
            def _cv(r):
                for g in range(d // 32):
                    a = fbuf[r, pl.ds(g * 32, _L)]
                    b = fbuf[r, pl.ds(g * 32 + _L, _L)]
                    ua = plsc.bitcast(a, jnp.uint32)
                    ub = plsc.bitcast(b, jnp.uint32)
                    ta = ua + jnp.uint32(0x7FFF) + ((ua >> 16) & 1)
                    tb = ub + jnp.uint32(0x7FFF) + ((ub >> 16) & 1)
                    w = (ta >> 16) | (tb & jnp.uint32(0xFFFF0000))
                    obuf[r, pl.ds(g * _L, _L)] = plsc.bitcast(w, jnp.int32)

            pltpu.sync_copy(obuf,
                            out_hbm.at[cid, pl.ds(r0 + q * cvr, cvr)])

    return pl.kernel(
        body,
        out_type=jax.ShapeDtypeStruct((_NC, _APAD, d // 2), jnp.int32),
        compiler_params=pltpu.CompilerParams(needs_layout_passes=False),
        mesh=mesh,
        scratch_types=[
            pltpu.VMEM((ch, _K), jnp.int32),
            pltpu.VMEM((ch, _K), jnp.int32),
            pltpu.VMEM((_K, d), _F32),
            pltpu.VMEM((cvr, d), _F32),
            pltpu.VMEM((cvr, d // 2), jnp.int32),
            pltpu.VMEM_SHARED((_APAD, d), _F32),
            pltpu.SemaphoreType.DMA,
        ],
    )


def _edge_agg_call(y, row3, col3):
    d = y.shape[1]
    ch = row3.shape[1]
    rpt = _APAD // _NS
    f = _edge_agg_kernel(d, ch)
    return f(y, row3, col3, jnp.zeros((rpt, d), _F32))


def kernel(x, edge_index, u, batch, params):
    n, d = x.shape
    e = edge_index.shape[1]
    row = edge_index[0].astype(jnp.int32)
    col = edge_index[1].astype(jnp.int32)

    npad = _NC * _R
    grp = _NS * _K
    ep = -(-e // grp) * grp

    row3 = jnp.concatenate([row, jnp.zeros((ep - e,), jnp.int32)])
    row3 = row3.reshape(_NS, ep // grp, _K)
    col3 = jnp.concatenate([col, jnp.full((ep - e,), n, jnp.int32)])
    col3 = col3.reshape(_NS, ep // grp, _K)

    xp = jnp.concatenate([x, jnp.zeros((npad - n, d), _F32)])
    batch2 = jnp.concatenate([batch.astype(jnp.int32),
                              jnp.full((npad - n,), 16, jnp.int32)])
    batch2 = batch2.reshape(npad, 1)

    hn = params[0]['n1_W2'].shape[1]

    def stack(f):
        return jnp.stack([f(p) for p in params])

    xs = (
        stack(lambda p: p['n1_W1']),
        stack(lambda p: p['n1_b1'].reshape(1, -1)),
        stack(lambda p: jnp.zeros((hn, d), _F32).at[:, :hn].set(p['n1_W2'])),
        stack(lambda p: jnp.zeros((1, d), _F32)
              .at[0, :hn].set(p['n1_b2']).at[0, hn].set(1.0)),
        stack(lambda p: p['n2_W1'][:d]),
        stack(lambda p: jnp.concatenate(
            [p['n2_W1'][d:d + hn],
             jnp.zeros((d - hn, hn), _F32)])[jnp.asarray(_PACK_LO)]),
        stack(lambda p: jnp.concatenate(
            [p['n2_W1'][d:d + hn],
             jnp.zeros((d - hn, hn), _F32)])[jnp.asarray(_PACK_HI)]),
        stack(lambda p: p['n2_W1'][d + hn:]),
        stack(lambda p: p['n2_b1'].reshape(1, -1)),
        stack(lambda p: p['n2_W2']),
        stack(lambda p: p['n2_b2'].reshape(1, -1)),
        stack(lambda p: p['g_W1'][:d]),
        stack(lambda p: p['g_W1'][d:]),
        stack(lambda p: p['g_b1'].reshape(1, -1)),
        stack(lambda p: p['g_W2']),
        stack(lambda p: p['g_b2'].reshape(1, -1)),
    )

    def layer(carry, pw):
        x, u = carry
        (w1, b1, w2p, b2p, w2x, w2axl, w2axh, w2u, nb1, w22, nb2,
         gw1u, gw1g, gb1, gw2, gb2) = pw
        y = _mlp1_call(x, w1, b1, w2p, b2p)
        accs = _edge_agg_call(y, row3, col3)
        x, gm_sum, bc = _mlp2_call(x, accs, batch2, u,
                                   w2x, w2axl, w2axh, w2u, nb1, w22, nb2)
        u = _global_call(u, gm_sum, bc, gw1u, gw1g, gb1, gw2, gb2)
        return (x, u), None

    (xp, u), _ = lax.scan(layer, (xp, u), xs)
    return (xp[:n], u)
